# Initial kernel scaffold; baseline (speedup 1.0000x reference)
import jax
import jax.numpy as jnp
from jax.experimental import pallas as pl


def kernel(H_lr, emb_table, W, b):
    raise NotImplementedError("write your pallas kernel here")


# speedup vs baseline: 1.1068x; 1.1068x over previous
import functools

import jax
import jax.numpy as jnp
from jax import lax
from jax.experimental import pallas as pl
from jax.experimental.pallas import tpu as pltpu
from jax.experimental.pallas import tpu_sc as plsc

_LANES = 16


def _g_body(h_ref, w_ref, b_ref, g_ref):
    g = jnp.mean(h_ref[...], axis=1)
    g_ref[...] = (
        lax.dot_general(g, w_ref[...], (((1,), (1,)), ((), ())),
                        precision=lax.Precision.HIGHEST)
        + b_ref[...]
    )


def _compute_g(H_lr, W, b):
    Bsz, _, D = H_lr.shape
    return pl.pallas_call(
        _g_body,
        out_shape=jax.ShapeDtypeStruct((Bsz, D), jnp.float32),
    )(H_lr, W, b.reshape(1, D))


@functools.lru_cache(maxsize=None)
def _make_sc_add(hr_n, Bsz, D):
    info = plsc.get_sparse_core_info()
    nc, ns = info.num_cores, info.num_subcores
    nw = nc * ns
    max_r = max(8, (96 * 1024) // (D * 4))
    R = 8
    for cand in range(8, max_r + 1, 8):
        if hr_n % cand == 0:
            R = cand
    nblk = hr_n // R
    nchunk = D // _LANES

    mesh = plsc.VectorSubcoreMesh(core_axis_name="c", subcore_axis_name="s")

    @functools.partial(
        pl.kernel,
        out_type=jax.ShapeDtypeStruct((Bsz, hr_n, D), jnp.float32),
        mesh=mesh,
        scratch_types=[
            pltpu.VMEM((Bsz, D), jnp.float32),
            pltpu.VMEM((R, D), jnp.float32),
            pltpu.VMEM((Bsz, R, D), jnp.float32),
        ],
    )
    def sc_add(table_hbm, g_hbm, out_hbm, g_v, in_v, out_v):
        wid = lax.axis_index("s") * nc + lax.axis_index("c")
        nblk_w = (nblk - wid + nw - 1) // nw
        pltpu.sync_copy(g_hbm, g_v)

        def blk_body(i, carry):
            row0 = (wid + i * nw) * R
            pltpu.sync_copy(table_hbm.at[pl.ds(row0, R)], in_v)
            for bb in range(Bsz):
                gs = [g_v[bb, pl.ds(_LANES * j, _LANES)] for j in range(nchunk)]

                def row_body(r, c, gs=gs, bb=bb):
                    for j in range(nchunk):
                        sl = pl.ds(_LANES * j, _LANES)
                        out_v[bb, r, sl] = in_v[r, sl] + gs[j]
                    return c

                lax.fori_loop(0, R, row_body, 0)
                pltpu.sync_copy(out_v.at[bb], out_hbm.at[bb, pl.ds(row0, R)])
            return carry

        lax.fori_loop(0, nblk_w, blk_body, 0)

    return sc_add


def kernel(H_lr, emb_table, W, b):
    hr_n, D = emb_table.shape
    Bsz = H_lr.shape[0]
    g = _compute_g(H_lr, W, b)
    return _make_sc_add(hr_n, Bsz, D)(emb_table, g)


# speedup vs baseline: 1.5905x; 1.4371x over previous
import functools

import jax
import jax.numpy as jnp
from jax import lax
from jax.experimental import pallas as pl
from jax.experimental.pallas import tpu as pltpu
from jax.experimental.pallas import tpu_sc as plsc

_LANES = 16


def _g_body(h_ref, w_ref, b_ref, g_ref):
    g = jnp.mean(h_ref[...], axis=1)
    g_ref[...] = (
        lax.dot_general(g, w_ref[...], (((1,), (1,)), ((), ())),
                        precision=lax.Precision.HIGHEST)
        + b_ref[...]
    )


def _compute_g(H_lr, W, b):
    Bsz, _, D = H_lr.shape
    return pl.pallas_call(
        _g_body,
        out_shape=jax.ShapeDtypeStruct((Bsz, D), jnp.float32),
    )(H_lr, W, b.reshape(1, D))


@functools.lru_cache(maxsize=None)
def _make_sc_add(hr_n, Bsz, D):
    info = plsc.get_sparse_core_info()
    nc, ns = info.num_cores, info.num_subcores
    nw = nc * ns
    max_r = max(8, (96 * 1024) // (D * 4))
    R = 8
    for cand in range(8, max_r + 1, 8):
        if hr_n % cand == 0:
            R = cand
    nblk = hr_n // R
    nbw = (nblk + nw - 1) // nw
    nchunk = D // _LANES

    mesh = plsc.VectorSubcoreMesh(core_axis_name="c", subcore_axis_name="s")

    @functools.partial(
        pl.kernel,
        out_type=jax.ShapeDtypeStruct((Bsz, hr_n, D), jnp.float32),
        mesh=mesh,
        scratch_types=[
            pltpu.VMEM((Bsz, D), jnp.float32),
            pltpu.VMEM((2, R, D), jnp.float32),
            pltpu.VMEM((Bsz, R, D), jnp.float32),
            pltpu.SemaphoreType.DMA,
            pltpu.SemaphoreType.DMA,
        ]
        + [pltpu.SemaphoreType.DMA for _ in range(Bsz)],
    )
    def sc_add(table_hbm, g_hbm, out_hbm, g_v, in_v, out_v, si0, si1, *so):
        wid = lax.axis_index("s") * nc + lax.axis_index("c")
        pltpu.sync_copy(g_hbm, g_v)
        sin = (si0, si1)

        def in_cp(t, slot):
            row0 = (wid + t * nw) * R
            return pltpu.make_async_copy(
                table_hbm.at[pl.ds(row0, R)], in_v.at[slot], sin[slot])

        def out_cp(t, bb):
            row0 = (wid + t * nw) * R
            return pltpu.make_async_copy(
                out_v.at[bb], out_hbm.at[bb, pl.ds(row0, R)], so[bb])

        def valid(t):
            return wid + t * nw < nblk

        @pl.when(valid(0))
        def _():
            in_cp(0, 0).start()

        for t in range(nbw):
            slot = t % 2

            @pl.when(valid(t))
            def _(t=t, slot=slot):
                if t + 1 < nbw:
                    @pl.when(valid(t + 1))
                    def _():
                        in_cp(t + 1, 1 - slot).start()
                in_cp(t, slot).wait()
                for bb in range(Bsz):
                    if t > 0:
                        out_cp(t - 1, bb).wait()
                    gs = [g_v[bb, pl.ds(_LANES * j, _LANES)]
                          for j in range(nchunk)]

                    def row_body(r, c, gs=gs, bb=bb, slot=slot):
                        for j in range(nchunk):
                            sl = pl.ds(_LANES * j, _LANES)
                            out_v[bb, r, sl] = in_v[slot, r, sl] + gs[j]
                        return c

                    lax.fori_loop(0, R, row_body, 0)
                    out_cp(t, bb).start()

        for bb in range(Bsz):
            out_cp(0, bb).wait()

    return sc_add


def kernel(H_lr, emb_table, W, b):
    hr_n, D = emb_table.shape
    Bsz = H_lr.shape[0]
    g = _compute_g(H_lr, W, b)
    return _make_sc_add(hr_n, Bsz, D)(emb_table, g)
